# Initial kernel scaffold; baseline (speedup 1.0000x reference)
import jax
import jax.numpy as jnp
from jax.experimental import pallas as pl


def kernel(u_emb, i_emb, a_emb, o_emb, s, ui_W0, ui_b0, ui_W1, ui_b1, ao_W0, ao_b0, ao_W1, ao_b1):
    raise NotImplementedError("write your pallas kernel here")


# speedup vs baseline: 1.3186x; 1.3186x over previous
import jax
import jax.numpy as jnp
from jax.experimental import pallas as pl

B = 4096
N = 50
D = 128
H1 = 256
OUT = 128
R = 3

BB = 64


def _leaky(x):
    return jnp.where(x > 0, x, 0.01 * x)


def _fused_body(u_ref, i_ref, a_ref, o_ref, s_ref,
                uw0u_ref, uw0i_ref, ub0_ref, uw1_ref, ub1_ref,
                aw0a_ref, aw0o_ref, ab0_ref, aw1_ref, ab1_ref,
                pred_ref):
    u = u_ref[...]
    i = i_ref[...]
    h_ui = _leaky(
        jnp.dot(u, uw0u_ref[...], preferred_element_type=jnp.float32)
        + jnp.dot(i, uw0i_ref[...], preferred_element_type=jnp.float32)
        + ub0_ref[...]
    )
    ue = _leaky(
        jnp.dot(h_ui, uw1_ref[...], preferred_element_type=jnp.float32)
        + ub1_ref[...]
    )

    xa = a_ref[...].reshape(BB * N, D)
    xo = o_ref[...].reshape(BB * N, D)
    s = s_ref[...]

    pred = jnp.zeros((BB, N), dtype=jnp.float32)
    for r in range(R):
        h = _leaky(
            jnp.dot(xa, aw0a_ref[r], preferred_element_type=jnp.float32)
            + jnp.dot(xo, aw0o_ref[r], preferred_element_type=jnp.float32)
            + ab0_ref[r]
        )
        out_r = _leaky(
            jnp.dot(h, aw1_ref[r], preferred_element_type=jnp.float32)
            + ab1_ref[r]
        )
        p_r = jnp.sum(
            out_r.reshape(BB, N, OUT) * ue[:, None, :], axis=-1
        )
        pred = pred + jnp.where(s == r, p_r, 0.0)
    pred_ref[...] = pred


def kernel(u_emb, i_emb, a_emb, o_emb, s, ui_W0, ui_b0, ui_W1, ui_b1,
           ao_W0, ao_b0, ao_W1, ao_b1):
    uw0u = ui_W0[:, :D].T
    uw0i = ui_W0[:, D:].T
    uw1 = ui_W1.T
    aw0a = ao_W0[:, :, :D].transpose(0, 2, 1)
    aw0o = ao_W0[:, :, D:].transpose(0, 2, 1)
    aw1 = ao_W1.transpose(0, 2, 1)
    s32 = s.astype(jnp.int32)

    grid = (B // BB,)

    def const(shape):
        nd = len(shape)
        return pl.BlockSpec(shape, lambda i: (0,) * nd)

    out = pl.pallas_call(
        _fused_body,
        grid=grid,
        in_specs=[
            pl.BlockSpec((BB, D), lambda i: (i, 0)),
            pl.BlockSpec((BB, D), lambda i: (i, 0)),
            pl.BlockSpec((BB, N, D), lambda i: (i, 0, 0)),
            pl.BlockSpec((BB, N, D), lambda i: (i, 0, 0)),
            pl.BlockSpec((BB, N), lambda i: (i, 0)),
            const((D, H1)), const((D, H1)), const((H1,)),
            const((H1, OUT)), const((OUT,)),
            const((R, D, H1)), const((R, D, H1)), const((R, H1)),
            const((R, H1, OUT)), const((R, OUT)),
        ],
        out_specs=pl.BlockSpec((BB, N), lambda i: (i, 0)),
        out_shape=jax.ShapeDtypeStruct((B, N), jnp.float32),
    )(u_emb, i_emb, a_emb, o_emb, s32,
      uw0u, uw0i, ui_b0, uw1, ui_b1,
      aw0a, aw0o, ao_b0, aw1, ao_b1)
    return out
